# baseline (speedup 1.0000x reference)
import functools

import jax
import jax.numpy as jnp
import numpy as np
from jax import lax
from jax.experimental import pallas as pl
from jax.experimental.pallas import tpu as pltpu
from jax.experimental.pallas import tpu_sc as plsc

N0 = 10000
N1 = 2500
E0 = 320000
E1 = 80000
D_IN = 128
C0 = 32
C1 = 64
K = 27
EPS = 1e-4

NCORES = 2
NSUB = 16
NW = NCORES * NSUB

W1P = 48
W2P = 80

CH0A, CH0B = 44, 116
TCH0 = NSUB * (CH0A + CH0B)
CH1A, CH1B = 8, 32
TCH1 = NSUB * (CH1A + CH1B)

ACC1_ROWS = 10240
ACC1_JUNK = 10000
ACC2_ROWS = 2560
ACC2_JUNK = 2500
NPOOL = 10240


def _edge_accum(ch_a, ch_b, width, acc_rows, name):
    rpt = acc_rows // NSUB
    nv = width // 16
    chmax = max(ch_a, ch_b)
    mesh = plsc.VectorSubcoreMesh(core_axis_name="c", subcore_axis_name="s")

    @functools.partial(
        pl.kernel,
        out_type=jax.ShapeDtypeStruct((NCORES, acc_rows, width), jnp.float32),
        mesh=mesh,
        scratch_types=[
            pltpu.VMEM((chmax, 128), jnp.int32),
            pltpu.VMEM((chmax, 128), jnp.int32),
            pltpu.VMEM((chmax, 128), jnp.int32),
            pltpu.VMEM((4, 128, width), jnp.float32),
            pltpu.VMEM((rpt, width), jnp.float32),
            pltpu.VMEM_SHARED((acc_rows, width), jnp.float32),
            [pltpu.SemaphoreType.DMA] * 4,
            [pltpu.SemaphoreType.DMA] * 4,
        ],
        compiler_params=pltpu.CompilerParams(use_tc_tiling_on_sc=False),
        name=name,
    )
    def k(table_hbm, src_hbm, kidx_hbm, dst_hbm, out_hbm,
          gidx_v, kidx_v, dst_v, rowbufs, stage, acc, gsems, ssems):
        c = lax.axis_index("c")
        s = lax.axis_index("s")
        base = jnp.where(c == 0, s * ch_a, NSUB * ch_a + s * ch_b)
        cnt = jnp.where(c == 0, ch_a, ch_b)

        zero16 = jnp.zeros((16,), jnp.float32)

        @pl.loop(0, rpt)
        def _zero(i):
            for v in range(nv):
                stage[i, pl.ds(v * 16, 16)] = zero16

        pltpu.sync_copy(stage, acc.at[pl.ds(s * rpt, rpt)])

        pltpu.sync_copy(src_hbm.at[pl.ds(base, chmax)], gidx_v)
        pltpu.sync_copy(kidx_hbm.at[pl.ds(base, chmax)], kidx_v)
        pltpu.sync_copy(dst_hbm.at[pl.ds(base, chmax)], dst_v)

        @pl.loop(0, chmax)
        def _fuse(j):
            for v in range(8):
                sl = pl.ds(v * 16, 16)
                gidx_v[j, sl] = gidx_v[j, sl] * K + kidx_v[j, sl]

        plsc.subcore_barrier()

        def _g(j, b):
            pltpu.async_copy(table_hbm.at[gidx_v.at[j]], rowbufs.at[b], gsems[b])

        def _gw(b):
            pltpu.make_async_copy(table_hbm.at[gidx_v.at[0]],
                                  rowbufs.at[b], gsems[b]).wait()

        def _s(j, b):
            pltpu.async_copy(rowbufs.at[b], acc.at[dst_v.at[j]], ssems[b],
                             add=True)

        def _sw(b):
            pltpu.make_async_copy(rowbufs.at[b], acc.at[dst_v.at[0]],
                                  ssems[b]).wait()

        nq = cnt // 4
        _g(0, 0)
        _g(1, 1)
        _gw(0); _s(0, 0); _g(2, 2)
        _gw(1); _s(1, 1); _g(3, 3)
        _gw(2); _s(2, 2); _sw(0); _g(4, 0)
        _gw(3); _s(3, 3); _sw(1); _g(5, 1)

        @pl.loop(1, nq)
        def _edges(i):
            j = i * 4
            for t in range(4):
                b2 = (t + 2) % 4
                _gw(t)
                _s(j + t, t)
                _sw(b2)
                if t < 2:
                    _g(j + t + 2, b2)
                else:
                    @pl.when(i < nq - 1)
                    def _pref(jj=j + t + 2, bb=b2):
                        _g(jj, bb)

        _sw(2)
        _sw(3)
        plsc.subcore_barrier()

        rsl = pl.ds(s * rpt, rpt)
        pltpu.sync_copy(acc.at[rsl], stage)
        pltpu.sync_copy(stage, out_hbm.at[c, rsl])

    return k


def _pool_accum():
    rpt = ACC2_ROWS // NSUB
    rows_per_worker = NPOOL // NW
    nv = W2P // 16
    mesh = plsc.VectorSubcoreMesh(core_axis_name="c", subcore_axis_name="s")

    @functools.partial(
        pl.kernel,
        out_type=jax.ShapeDtypeStruct((NCORES, ACC2_ROWS, W2P), jnp.float32),
        mesh=mesh,
        scratch_types=[
            pltpu.VMEM((5, 64), jnp.int32),
            pltpu.VMEM((rows_per_worker, W2P), jnp.float32),
            pltpu.VMEM((rpt, W2P), jnp.float32),
            pltpu.VMEM_SHARED((ACC2_ROWS, W2P), jnp.float32),
        ],
        compiler_params=pltpu.CompilerParams(use_tc_tiling_on_sc=False),
        name="sc_pool_accum",
    )
    def k(d_hbm, pm_hbm, out_hbm, pm_v, dbuf, stage, acc):
        c = lax.axis_index("c")
        s = lax.axis_index("s")
        wid = c * NSUB + s

        zero16 = jnp.zeros((16,), jnp.float32)

        @pl.loop(0, rpt)
        def _zero(i):
            for v in range(nv):
                stage[i, pl.ds(v * 16, 16)] = zero16

        pltpu.sync_copy(stage, acc.at[pl.ds(s * rpt, rpt)])
        pltpu.sync_copy(pm_hbm.at[wid], pm_v)
        pltpu.sync_copy(d_hbm.at[wid], dbuf)
        plsc.subcore_barrier()

        for t in range(5):
            pltpu.sync_copy(dbuf.at[pl.ds(t * 64, 64)], acc.at[pm_v.at[t]], add=True)

        plsc.subcore_barrier()
        rsl = pl.ds(s * rpt, rpt)
        pltpu.sync_copy(acc.at[rsl], stage)
        pltpu.sync_copy(stage, out_hbm.at[c, rsl])

    return k


def _unpool_gather():
    rows_per_worker = NPOOL // NW
    mesh = plsc.VectorSubcoreMesh(core_axis_name="c", subcore_axis_name="s")

    @functools.partial(
        pl.kernel,
        out_type=jax.ShapeDtypeStruct((NPOOL, C0), jnp.float32),
        mesh=mesh,
        scratch_types=[
            pltpu.VMEM((4, 80), jnp.int32),
            pltpu.VMEM((rows_per_worker, C0), jnp.float32),
            pltpu.SemaphoreType.DMA,
        ],
        compiler_params=pltpu.CompilerParams(use_tc_tiling_on_sc=False),
        name="sc_unpool_gather",
    )
    def k(u_hbm, pm_hbm, out_hbm, pm_v, gbuf, sem):
        c = lax.axis_index("c")
        s = lax.axis_index("s")
        wid = c * NSUB + s

        pltpu.sync_copy(pm_hbm.at[wid], pm_v)
        for t in range(4):
            pltpu.async_copy(u_hbm.at[pm_v.at[t]],
                             gbuf.at[pl.ds(t * 80, 80)], sem).wait()
        pltpu.sync_copy(gbuf, out_hbm.at[pl.ds(wid * rows_per_worker,
                                               rows_per_worker)])

    return k


def _lift_bn_relu(x, w, g, b):
    def body(x_ref, w_ref, g_ref, b_ref, h_ref):
        feat = jnp.dot(x_ref[...], w_ref[...], preferred_element_type=jnp.float32)
        mean = jnp.mean(feat, axis=0, keepdims=True)
        var = jnp.mean((feat - mean) ** 2, axis=0, keepdims=True)
        hn = g_ref[...] * (feat - mean) / jnp.sqrt(var + EPS) + b_ref[...]
        h_ref[...] = jnp.maximum(hn, 0.0)

    n = x.shape[0]
    co = w.shape[1]
    return pl.pallas_call(
        body, out_shape=jax.ShapeDtypeStruct((n, co), jnp.float32),
    )(x, w, g.reshape(1, -1), b.reshape(1, -1))


def _build_table(h, wcat, onesrow, bm):
    n, cin = h.shape
    wn = wcat.shape[1]

    def body(h_ref, w_ref, o_ref, out_ref):
        out_ref[...] = (jnp.dot(h_ref[...], w_ref[...],
                                preferred_element_type=jnp.float32) + o_ref[...])

    return pl.pallas_call(
        body,
        grid=(n // bm,),
        in_specs=[pl.BlockSpec((bm, cin), lambda i: (i, 0)),
                  pl.BlockSpec((cin, wn), lambda i: (0, 0)),
                  pl.BlockSpec((1, wn), lambda i: (0, 0))],
        out_specs=pl.BlockSpec((bm, wn), lambda i: (i, 0)),
        out_shape=jax.ShapeDtypeStruct((n, wn), jnp.float32),
    )(h, wcat, onesrow)


def _conv1_norm_down(p0, p1, dw):
    def body(p0_ref, p1_ref, dw_ref, skip_ref, dext_ref):
        acc = p0_ref[...] + p1_ref[...]
        num = acc[:N0, :C0]
        deg = acc[:N0, C0:C0 + 1]
        skip = num / jnp.maximum(deg, 1.0)
        skip_ref[...] = skip
        d = jnp.dot(jnp.maximum(skip, 0.0), dw_ref[...],
                    preferred_element_type=jnp.float32)
        dfull = jnp.concatenate(
            [d, jnp.ones((N0, 1), jnp.float32),
             jnp.zeros((N0, W2P - C1 - 1), jnp.float32)], axis=1)
        dext_ref[...] = jnp.concatenate(
            [dfull, jnp.zeros((NPOOL - N0, W2P), jnp.float32)], axis=0)

    return pl.pallas_call(
        body,
        out_shape=(jax.ShapeDtypeStruct((N0, C0), jnp.float32),
                   jax.ShapeDtypeStruct((NPOOL, W2P), jnp.float32)),
    )(p0, p1, dw)


def _pool_bn_relu(p0, p1, g, b):
    def body(p0_ref, p1_ref, g_ref, b_ref, h_ref):
        acc = p0_ref[...] + p1_ref[...]
        pooled = acc[:, :C1] / jnp.maximum(acc[:, C1:C1 + 1], 1.0)
        mean = jnp.mean(pooled, axis=0, keepdims=True)
        var = jnp.mean((pooled - mean) ** 2, axis=0, keepdims=True)
        hn = g_ref[...] * (pooled - mean) / jnp.sqrt(var + EPS) + b_ref[...]
        h_ref[...] = jnp.maximum(hn, 0.0)

    return pl.pallas_call(
        body, out_shape=jax.ShapeDtypeStruct((N1, C1), jnp.float32),
    )(p0, p1, g.reshape(1, -1), b.reshape(1, -1))


def _conv2_norm_up(p0, p1, uw):
    def body(p0_ref, p1_ref, uw_ref, u_ref):
        acc = p0_ref[...] + p1_ref[...]
        h2 = acc[:, :C1] / jnp.maximum(acc[:, C1:C1 + 1], 1.0)
        u_ref[...] = jnp.dot(h2, uw_ref[...], preferred_element_type=jnp.float32)

    return pl.pallas_call(
        body, out_shape=jax.ShapeDtypeStruct((N1, C0), jnp.float32),
    )(p0, p1, uw)


def _join_out(skip, up, wa, wb):
    def body(s_ref, u_ref, wa_ref, wb_ref, out_ref):
        out_ref[...] = (
            jnp.dot(jnp.maximum(s_ref[...], 0.0), wa_ref[...],
                    preferred_element_type=jnp.float32)
            + jnp.dot(jnp.maximum(u_ref[...], 0.0), wb_ref[...],
                      preferred_element_type=jnp.float32))

    return pl.pallas_call(
        body, out_shape=jax.ShapeDtypeStruct((N0, C0), jnp.float32),
    )(skip, up, wa, wb)


def _make_wcat(conv_w, width):
    kk, cin, cout = conv_w.shape
    wp = jnp.pad(conv_w, ((0, 0), (0, 0), (0, width - cout)))
    return jnp.transpose(wp, (1, 0, 2)).reshape(cin, kk * width)


def _ones_row(width, cout):
    row = np.zeros((1, K * width), np.float32)
    row[0, np.arange(K) * width + cout] = 1.0
    return jnp.asarray(row)


@functools.lru_cache(maxsize=None)
def _sc_kernels():
    return (_edge_accum(CH0A, CH0B, W1P, ACC1_ROWS, "sc_edge_accum1"),
            _edge_accum(CH1A, CH1B, W2P, ACC2_ROWS, "sc_edge_accum2"),
            _pool_accum(),
            _unpool_gather())


def _pad_edges(src, kidx, dst, tot_chunks, junk, junk_rows):
    pad = tot_chunks * 128 - src.shape[0]
    jd = junk + (np.arange(pad, dtype=np.int32) % junk_rows)
    srcp = jnp.concatenate([src, jnp.zeros((pad,), jnp.int32)]).reshape(tot_chunks, 128)
    kip = jnp.concatenate([kidx, jnp.zeros((pad,), jnp.int32)]).reshape(tot_chunks, 128)
    dstp = jnp.concatenate([dst, jnp.asarray(jd)]).reshape(tot_chunks, 128)
    return srcp, kip, dstp


def kernel(x, edge_index, kernel_idx, pool_map, edge_index2, kernel_idx2,
           lin0_W, conv1_W, bn1_g, bn1_b, down_W, conv2_W, bn2_g, bn2_b,
           up_W, out_W):
    src0, dst0 = edge_index[0], edge_index[1]
    src1, dst1 = edge_index2[0], edge_index2[1]
    _edge_accum1, _edge_accum2, _pool_accum_k, _unpool_k = _sc_kernels()

    w1cat = _make_wcat(conv1_W, W1P)
    o1 = _ones_row(W1P, C0)
    w2cat = _make_wcat(conv2_W, W2P)
    o2 = _ones_row(W2P, C1)
    s0p, k0p, d0p = _pad_edges(src0, kernel_idx, dst0, TCH0, ACC1_JUNK, ACC1_ROWS - ACC1_JUNK)
    s1p, k1p, d1p = _pad_edges(src1, kernel_idx2, dst1, TCH1, ACC2_JUNK, ACC2_ROWS - ACC2_JUNK)
    pmp = jnp.concatenate([pool_map, jnp.zeros((NPOOL - N0,), jnp.int32)])
    pm_pool = pmp.reshape(NW, 5, 64)
    pm_up = pmp.reshape(NW, 4, 80)

    h = _lift_bn_relu(x, lin0_W, bn1_g, bn1_b)
    table1 = _build_table(h, w1cat, o1, 1000).reshape(N0 * K, W1P)
    acc1 = _edge_accum1(table1, s0p, k0p, d0p)
    skip, d_ext = _conv1_norm_down(acc1[0], acc1[1], down_W)

    d_rows = d_ext.reshape(NW, NPOOL // NW, W2P)
    paccs = _pool_accum_k(d_rows, pm_pool)
    h2p = _pool_bn_relu(paccs[0, :N1], paccs[1, :N1], bn2_g, bn2_b)

    h2p_pad = jnp.concatenate([h2p, jnp.zeros((2560 - N1, C1), jnp.float32)])
    table2 = _build_table(h2p_pad, w2cat, o2, 512).reshape(2560 * K, W2P)
    acc2 = _edge_accum2(table2, s1p, k1p, d1p)
    u = _conv2_norm_up(acc2[0, :N1], acc2[1, :N1], up_W)

    up = _unpool_k(u, pm_up)[:N0]
    out = _join_out(skip, up, out_W[:C0], out_W[C0:])
    return out


# speedup vs baseline: 1.0623x; 1.0623x over previous
import functools

import jax
import jax.numpy as jnp
import numpy as np
from jax import lax
from jax.experimental import pallas as pl
from jax.experimental.pallas import tpu as pltpu
from jax.experimental.pallas import tpu_sc as plsc

N0 = 10000
N1 = 2500
E0 = 320000
E1 = 80000
D_IN = 128
C0 = 32
C1 = 64
K = 27
EPS = 1e-4

NCORES = 2
NSUB = 16
NW = NCORES * NSUB

W1P = 48
W2P = 80

CH0A, CH0B = 116, 44
TCH0 = NSUB * (CH0A + CH0B)
CH1A, CH1B = 32, 8
TCH1 = NSUB * (CH1A + CH1B)

ACC1_ROWS = 10240
ACC1_JUNK = 10000
ACC2_ROWS = 2560
ACC2_JUNK = 2500
NPOOL = 10240


def _edge_accum(ch_a, ch_b, width, acc_rows, name):
    rpt = acc_rows // NSUB
    nv = width // 16
    chmax = max(ch_a, ch_b)
    mesh = plsc.VectorSubcoreMesh(core_axis_name="c", subcore_axis_name="s")

    @functools.partial(
        pl.kernel,
        out_type=jax.ShapeDtypeStruct((NCORES, acc_rows, width), jnp.float32),
        mesh=mesh,
        scratch_types=[
            pltpu.VMEM((chmax, 128), jnp.int32),
            pltpu.VMEM((chmax, 128), jnp.int32),
            pltpu.VMEM((chmax, 128), jnp.int32),
            pltpu.VMEM((4, 128, width), jnp.float32),
            pltpu.VMEM((rpt, width), jnp.float32),
            pltpu.VMEM_SHARED((acc_rows, width), jnp.float32),
            [pltpu.SemaphoreType.DMA] * 4,
            [pltpu.SemaphoreType.DMA] * 4,
        ],
        compiler_params=pltpu.CompilerParams(use_tc_tiling_on_sc=False),
        name=name,
    )
    def k(table_hbm, src_hbm, kidx_hbm, dst_hbm, out_hbm,
          gidx_v, kidx_v, dst_v, rowbufs, stage, acc, gsems, ssems):
        c = lax.axis_index("c")
        s = lax.axis_index("s")
        base = jnp.where(c == 0, s * ch_a, NSUB * ch_a + s * ch_b)
        cnt = jnp.where(c == 0, ch_a, ch_b)

        zero16 = jnp.zeros((16,), jnp.float32)

        @pl.loop(0, rpt)
        def _zero(i):
            for v in range(nv):
                stage[i, pl.ds(v * 16, 16)] = zero16

        pltpu.sync_copy(stage, acc.at[pl.ds(s * rpt, rpt)])

        pltpu.sync_copy(src_hbm.at[pl.ds(base, chmax)], gidx_v)
        pltpu.sync_copy(kidx_hbm.at[pl.ds(base, chmax)], kidx_v)
        pltpu.sync_copy(dst_hbm.at[pl.ds(base, chmax)], dst_v)

        @pl.loop(0, chmax)
        def _fuse(j):
            for v in range(8):
                sl = pl.ds(v * 16, 16)
                gidx_v[j, sl] = gidx_v[j, sl] * K + kidx_v[j, sl]

        plsc.subcore_barrier()

        def _g(j, b):
            pltpu.async_copy(table_hbm.at[gidx_v.at[j]], rowbufs.at[b], gsems[b])

        def _gw(b):
            pltpu.make_async_copy(table_hbm.at[gidx_v.at[0]],
                                  rowbufs.at[b], gsems[b]).wait()

        def _s(j, b):
            pltpu.async_copy(rowbufs.at[b], acc.at[dst_v.at[j]], ssems[b],
                             add=True)

        def _sw(b):
            pltpu.make_async_copy(rowbufs.at[b], acc.at[dst_v.at[0]],
                                  ssems[b]).wait()

        nq = cnt // 4
        _g(0, 0)
        _g(1, 1)
        _gw(0); _s(0, 0); _g(2, 2)
        _gw(1); _s(1, 1); _g(3, 3)
        _gw(2); _s(2, 2); _sw(0); _g(4, 0)
        _gw(3); _s(3, 3); _sw(1); _g(5, 1)

        @pl.loop(1, nq)
        def _edges(i):
            j = i * 4
            for t in range(4):
                b2 = (t + 2) % 4
                _gw(t)
                _s(j + t, t)
                _sw(b2)
                if t < 2:
                    _g(j + t + 2, b2)
                else:
                    @pl.when(i < nq - 1)
                    def _pref(jj=j + t + 2, bb=b2):
                        _g(jj, bb)

        _sw(2)
        _sw(3)
        plsc.subcore_barrier()

        rsl = pl.ds(s * rpt, rpt)
        pltpu.sync_copy(acc.at[rsl], stage)
        pltpu.sync_copy(stage, out_hbm.at[c, rsl])

    return k


def _pool_accum():
    rpt = ACC2_ROWS // NSUB
    rows_per_worker = NPOOL // NW
    nv = W2P // 16
    mesh = plsc.VectorSubcoreMesh(core_axis_name="c", subcore_axis_name="s")

    @functools.partial(
        pl.kernel,
        out_type=jax.ShapeDtypeStruct((NCORES, ACC2_ROWS, W2P), jnp.float32),
        mesh=mesh,
        scratch_types=[
            pltpu.VMEM((5, 64), jnp.int32),
            pltpu.VMEM((rows_per_worker, W2P), jnp.float32),
            pltpu.VMEM((rpt, W2P), jnp.float32),
            pltpu.VMEM_SHARED((ACC2_ROWS, W2P), jnp.float32),
        ],
        compiler_params=pltpu.CompilerParams(use_tc_tiling_on_sc=False),
        name="sc_pool_accum",
    )
    def k(d_hbm, pm_hbm, out_hbm, pm_v, dbuf, stage, acc):
        c = lax.axis_index("c")
        s = lax.axis_index("s")
        wid = c * NSUB + s

        zero16 = jnp.zeros((16,), jnp.float32)

        @pl.loop(0, rpt)
        def _zero(i):
            for v in range(nv):
                stage[i, pl.ds(v * 16, 16)] = zero16

        pltpu.sync_copy(stage, acc.at[pl.ds(s * rpt, rpt)])
        pltpu.sync_copy(pm_hbm.at[wid], pm_v)
        pltpu.sync_copy(d_hbm.at[wid], dbuf)
        plsc.subcore_barrier()

        for t in range(5):
            pltpu.sync_copy(dbuf.at[pl.ds(t * 64, 64)], acc.at[pm_v.at[t]], add=True)

        plsc.subcore_barrier()
        rsl = pl.ds(s * rpt, rpt)
        pltpu.sync_copy(acc.at[rsl], stage)
        pltpu.sync_copy(stage, out_hbm.at[c, rsl])

    return k


def _unpool_gather():
    rows_per_worker = NPOOL // NW
    mesh = plsc.VectorSubcoreMesh(core_axis_name="c", subcore_axis_name="s")

    @functools.partial(
        pl.kernel,
        out_type=jax.ShapeDtypeStruct((NPOOL, C0), jnp.float32),
        mesh=mesh,
        scratch_types=[
            pltpu.VMEM((4, 80), jnp.int32),
            pltpu.VMEM((rows_per_worker, C0), jnp.float32),
            pltpu.SemaphoreType.DMA,
        ],
        compiler_params=pltpu.CompilerParams(use_tc_tiling_on_sc=False),
        name="sc_unpool_gather",
    )
    def k(u_hbm, pm_hbm, out_hbm, pm_v, gbuf, sem):
        c = lax.axis_index("c")
        s = lax.axis_index("s")
        wid = c * NSUB + s

        pltpu.sync_copy(pm_hbm.at[wid], pm_v)
        for t in range(4):
            pltpu.async_copy(u_hbm.at[pm_v.at[t]],
                             gbuf.at[pl.ds(t * 80, 80)], sem).wait()
        pltpu.sync_copy(gbuf, out_hbm.at[pl.ds(wid * rows_per_worker,
                                               rows_per_worker)])

    return k


def _lift_bn_relu(x, w, g, b):
    def body(x_ref, w_ref, g_ref, b_ref, h_ref):
        feat = jnp.dot(x_ref[...], w_ref[...], preferred_element_type=jnp.float32)
        mean = jnp.mean(feat, axis=0, keepdims=True)
        var = jnp.mean((feat - mean) ** 2, axis=0, keepdims=True)
        hn = g_ref[...] * (feat - mean) / jnp.sqrt(var + EPS) + b_ref[...]
        h_ref[...] = jnp.maximum(hn, 0.0)

    n = x.shape[0]
    co = w.shape[1]
    return pl.pallas_call(
        body, out_shape=jax.ShapeDtypeStruct((n, co), jnp.float32),
    )(x, w, g.reshape(1, -1), b.reshape(1, -1))


def _build_table(h, wcat, onesrow, bm):
    n, cin = h.shape
    wn = wcat.shape[1]

    def body(h_ref, w_ref, o_ref, out_ref):
        out_ref[...] = (jnp.dot(h_ref[...], w_ref[...],
                                preferred_element_type=jnp.float32) + o_ref[...])

    return pl.pallas_call(
        body,
        grid=(n // bm,),
        in_specs=[pl.BlockSpec((bm, cin), lambda i: (i, 0)),
                  pl.BlockSpec((cin, wn), lambda i: (0, 0)),
                  pl.BlockSpec((1, wn), lambda i: (0, 0))],
        out_specs=pl.BlockSpec((bm, wn), lambda i: (i, 0)),
        out_shape=jax.ShapeDtypeStruct((n, wn), jnp.float32),
    )(h, wcat, onesrow)


def _conv1_norm_down(p0, p1, dw):
    def body(p0_ref, p1_ref, dw_ref, skip_ref, dext_ref):
        acc = p0_ref[...] + p1_ref[...]
        num = acc[:N0, :C0]
        deg = acc[:N0, C0:C0 + 1]
        skip = num / jnp.maximum(deg, 1.0)
        skip_ref[...] = skip
        d = jnp.dot(jnp.maximum(skip, 0.0), dw_ref[...],
                    preferred_element_type=jnp.float32)
        dfull = jnp.concatenate(
            [d, jnp.ones((N0, 1), jnp.float32),
             jnp.zeros((N0, W2P - C1 - 1), jnp.float32)], axis=1)
        dext_ref[...] = jnp.concatenate(
            [dfull, jnp.zeros((NPOOL - N0, W2P), jnp.float32)], axis=0)

    return pl.pallas_call(
        body,
        out_shape=(jax.ShapeDtypeStruct((N0, C0), jnp.float32),
                   jax.ShapeDtypeStruct((NPOOL, W2P), jnp.float32)),
    )(p0, p1, dw)


def _pool_bn_relu(p0, p1, g, b):
    def body(p0_ref, p1_ref, g_ref, b_ref, h_ref):
        acc = p0_ref[...] + p1_ref[...]
        pooled = acc[:, :C1] / jnp.maximum(acc[:, C1:C1 + 1], 1.0)
        mean = jnp.mean(pooled, axis=0, keepdims=True)
        var = jnp.mean((pooled - mean) ** 2, axis=0, keepdims=True)
        hn = g_ref[...] * (pooled - mean) / jnp.sqrt(var + EPS) + b_ref[...]
        h_ref[...] = jnp.maximum(hn, 0.0)

    return pl.pallas_call(
        body, out_shape=jax.ShapeDtypeStruct((N1, C1), jnp.float32),
    )(p0, p1, g.reshape(1, -1), b.reshape(1, -1))


def _conv2_norm_up(p0, p1, uw):
    def body(p0_ref, p1_ref, uw_ref, u_ref):
        acc = p0_ref[...] + p1_ref[...]
        h2 = acc[:, :C1] / jnp.maximum(acc[:, C1:C1 + 1], 1.0)
        u_ref[...] = jnp.dot(h2, uw_ref[...], preferred_element_type=jnp.float32)

    return pl.pallas_call(
        body, out_shape=jax.ShapeDtypeStruct((N1, C0), jnp.float32),
    )(p0, p1, uw)


def _join_out(skip, up, wa, wb):
    def body(s_ref, u_ref, wa_ref, wb_ref, out_ref):
        out_ref[...] = (
            jnp.dot(jnp.maximum(s_ref[...], 0.0), wa_ref[...],
                    preferred_element_type=jnp.float32)
            + jnp.dot(jnp.maximum(u_ref[...], 0.0), wb_ref[...],
                      preferred_element_type=jnp.float32))

    return pl.pallas_call(
        body, out_shape=jax.ShapeDtypeStruct((N0, C0), jnp.float32),
    )(skip, up, wa, wb)


def _make_wcat(conv_w, width):
    kk, cin, cout = conv_w.shape
    wp = jnp.pad(conv_w, ((0, 0), (0, 0), (0, width - cout)))
    return jnp.transpose(wp, (1, 0, 2)).reshape(cin, kk * width)


def _ones_row(width, cout):
    row = np.zeros((1, K * width), np.float32)
    row[0, np.arange(K) * width + cout] = 1.0
    return jnp.asarray(row)


@functools.lru_cache(maxsize=None)
def _sc_kernels():
    return (_edge_accum(CH0A, CH0B, W1P, ACC1_ROWS, "sc_edge_accum1"),
            _edge_accum(CH1A, CH1B, W2P, ACC2_ROWS, "sc_edge_accum2"),
            _pool_accum(),
            _unpool_gather())


def _pad_edges(src, kidx, dst, tot_chunks, junk, junk_rows):
    pad = tot_chunks * 128 - src.shape[0]
    jd = junk + (np.arange(pad, dtype=np.int32) % junk_rows)
    srcp = jnp.concatenate([src, jnp.zeros((pad,), jnp.int32)]).reshape(tot_chunks, 128)
    kip = jnp.concatenate([kidx, jnp.zeros((pad,), jnp.int32)]).reshape(tot_chunks, 128)
    dstp = jnp.concatenate([dst, jnp.asarray(jd)]).reshape(tot_chunks, 128)
    return srcp, kip, dstp


def kernel(x, edge_index, kernel_idx, pool_map, edge_index2, kernel_idx2,
           lin0_W, conv1_W, bn1_g, bn1_b, down_W, conv2_W, bn2_g, bn2_b,
           up_W, out_W):
    src0, dst0 = edge_index[0], edge_index[1]
    src1, dst1 = edge_index2[0], edge_index2[1]
    _edge_accum1, _edge_accum2, _pool_accum_k, _unpool_k = _sc_kernels()

    w1cat = _make_wcat(conv1_W, W1P)
    o1 = _ones_row(W1P, C0)
    w2cat = _make_wcat(conv2_W, W2P)
    o2 = _ones_row(W2P, C1)
    s0p, k0p, d0p = _pad_edges(src0, kernel_idx, dst0, TCH0, ACC1_JUNK, ACC1_ROWS - ACC1_JUNK)
    s1p, k1p, d1p = _pad_edges(src1, kernel_idx2, dst1, TCH1, ACC2_JUNK, ACC2_ROWS - ACC2_JUNK)
    pmp = jnp.concatenate([pool_map, jnp.zeros((NPOOL - N0,), jnp.int32)])
    pm_pool = pmp.reshape(NW, 5, 64)
    pm_up = pmp.reshape(NW, 4, 80)

    h = _lift_bn_relu(x, lin0_W, bn1_g, bn1_b)
    table1 = _build_table(h, w1cat, o1, 1000).reshape(N0 * K, W1P)
    acc1 = _edge_accum1(table1, s0p, k0p, d0p)
    skip, d_ext = _conv1_norm_down(acc1[0], acc1[1], down_W)

    d_rows = d_ext.reshape(NW, NPOOL // NW, W2P)
    paccs = _pool_accum_k(d_rows, pm_pool)
    h2p = _pool_bn_relu(paccs[0, :N1], paccs[1, :N1], bn2_g, bn2_b)

    h2p_pad = jnp.concatenate([h2p, jnp.zeros((2560 - N1, C1), jnp.float32)])
    table2 = _build_table(h2p_pad, w2cat, o2, 512).reshape(2560 * K, W2P)
    acc2 = _edge_accum2(table2, s1p, k1p, d1p)
    u = _conv2_norm_up(acc2[0, :N1], acc2[1, :N1], up_W)

    up = _unpool_k(u, pm_up)[:N0]
    out = _join_out(skip, up, out_W[:C0], out_W[C0:])
    return out


# speedup vs baseline: 1.2910x; 1.2153x over previous
import functools

import jax
import jax.numpy as jnp
import numpy as np
from jax import lax
from jax.experimental import pallas as pl
from jax.experimental.pallas import tpu as pltpu
from jax.experimental.pallas import tpu_sc as plsc

N0 = 10000
N1 = 2500
E0 = 320000
E1 = 80000
D_IN = 128
C0 = 32
C1 = 64
K = 27
EPS = 1e-4

NCORES = 2
NSUB = 16
NW = NCORES * NSUB

W1P = 32
W2P = 64
WPOOL = 80

CH0A, CH0B = 116, 44
TCH0 = NSUB * (CH0A + CH0B)
CH1A, CH1B = 32, 8
TCH1 = NSUB * (CH1A + CH1B)

ACC1_ROWS = 10240
ACC1_JUNK = 10000
ACC2_ROWS = 2560
ACC2_JUNK = 2500
NPOOL = 10240


def _edge_accum(ch_a, ch_b, width, acc_rows, name):
    rpt = acc_rows // NSUB
    nv = width // 16
    chmax = max(ch_a, ch_b)
    mesh = plsc.VectorSubcoreMesh(core_axis_name="c", subcore_axis_name="s")

    @functools.partial(
        pl.kernel,
        out_type=(jax.ShapeDtypeStruct((NCORES, acc_rows, width), jnp.float32),
                  jax.ShapeDtypeStruct((NCORES, acc_rows), jnp.float32)),
        mesh=mesh,
        scratch_types=[
            pltpu.VMEM((chmax, 128), jnp.int32),
            pltpu.VMEM((chmax, 128), jnp.int32),
            pltpu.VMEM((chmax, 128), jnp.int32),
            pltpu.VMEM((4, 128, width), jnp.float32),
            pltpu.VMEM((rpt, width), jnp.float32),
            pltpu.VMEM((acc_rows,), jnp.float32),
            pltpu.VMEM((rpt,), jnp.float32),
            pltpu.VMEM((rpt,), jnp.float32),
            pltpu.VMEM_SHARED((acc_rows, width), jnp.float32),
            pltpu.VMEM_SHARED((NSUB, acc_rows), jnp.float32),
            [pltpu.SemaphoreType.DMA] * 4,
            [pltpu.SemaphoreType.DMA] * 4,
        ],
        compiler_params=pltpu.CompilerParams(use_tc_tiling_on_sc=False,
                                             needs_layout_passes=False),
        name=name,
    )
    def k(table_hbm, src_hbm, kidx_hbm, dst_hbm, out_hbm, deg_hbm,
          gidx_v, kidx_v, dst_v, rowbufs, stage, deg_v, dacc_v, dtmp_v,
          acc, degs, gsems, ssems):
        c = lax.axis_index("c")
        s = lax.axis_index("s")
        base = jnp.where(c == 0, s * ch_a, NSUB * ch_a + s * ch_b)
        cnt = jnp.where(c == 0, ch_a, ch_b)

        zero16 = jnp.zeros((16,), jnp.float32)
        ones16 = jnp.ones((16,), jnp.float32)

        @pl.loop(0, rpt)
        def _zero(i):
            for v in range(nv):
                stage[i, pl.ds(v * 16, 16)] = zero16

        pltpu.sync_copy(stage, acc.at[pl.ds(s * rpt, rpt)])

        @pl.loop(0, acc_rows // 16)
        def _zerod(i):
            deg_v[pl.ds(i * 16, 16)] = zero16

        pltpu.sync_copy(src_hbm.at[pl.ds(base, chmax)], gidx_v)
        pltpu.sync_copy(kidx_hbm.at[pl.ds(base, chmax)], kidx_v)
        pltpu.sync_copy(dst_hbm.at[pl.ds(base, chmax)], dst_v)

        @pl.loop(0, chmax)
        def _fuse(j):
            for v in range(8):
                sl = pl.ds(v * 16, 16)
                gidx_v[j, sl] = gidx_v[j, sl] * K + kidx_v[j, sl]

        plsc.subcore_barrier()

        def _g(j, b):
            pltpu.async_copy(table_hbm.at[gidx_v.at[j]], rowbufs.at[b], gsems[b])

        def _gw(b):
            pltpu.make_async_copy(table_hbm.at[gidx_v.at[0]],
                                  rowbufs.at[b], gsems[b]).wait()

        def _s(j, b):
            pltpu.async_copy(rowbufs.at[b], acc.at[dst_v.at[j]], ssems[b],
                             add=True)

        def _sw(b):
            pltpu.make_async_copy(rowbufs.at[b], acc.at[dst_v.at[0]],
                                  ssems[b]).wait()

        def _hist(j):
            for v in range(8):
                idx = dst_v[j, pl.ds(v * 16, 16)]
                plsc.addupdate_scatter(deg_v, [idx], ones16)

        nq = cnt // 4
        _g(0, 0)
        _g(1, 1)
        _gw(0); _s(0, 0); _g(2, 2); _hist(0)
        _gw(1); _s(1, 1); _g(3, 3); _hist(1)
        _gw(2); _s(2, 2); _sw(0); _g(4, 0); _hist(2)
        _gw(3); _s(3, 3); _sw(1); _g(5, 1); _hist(3)

        @pl.loop(1, nq)
        def _edges(i):
            j = i * 4
            for t in range(4):
                b2 = (t + 2) % 4
                _gw(t)
                _s(j + t, t)
                _sw(b2)
                if t < 2:
                    _g(j + t + 2, b2)
                else:
                    @pl.when(i < nq - 1)
                    def _pref(jj=j + t + 2, bb=b2):
                        _g(jj, bb)
                _hist(j + t)

        _sw(2)
        _sw(3)
        pltpu.sync_copy(deg_v, degs.at[s])
        plsc.subcore_barrier()

        rsl = pl.ds(s * rpt, rpt)
        @pl.loop(0, rpt // 16)
        def _zeroa(i):
            dacc_v[pl.ds(i * 16, 16)] = zero16

        for t2 in range(NSUB):
            pltpu.sync_copy(degs.at[t2, rsl], dtmp_v)

            @pl.loop(0, rpt // 16)
            def _red(i):
                sl = pl.ds(i * 16, 16)
                dacc_v[sl] = dacc_v[sl] + dtmp_v[sl]

        pltpu.sync_copy(dacc_v, deg_hbm.at[c, rsl])
        pltpu.sync_copy(acc.at[rsl], stage)
        pltpu.sync_copy(stage, out_hbm.at[c, rsl])

    return k


def _pool_accum():
    rpt = ACC2_ROWS // NSUB
    rows_per_worker = NPOOL // NW
    nv = WPOOL // 16
    mesh = plsc.VectorSubcoreMesh(core_axis_name="c", subcore_axis_name="s")

    @functools.partial(
        pl.kernel,
        out_type=jax.ShapeDtypeStruct((NCORES, ACC2_ROWS, WPOOL), jnp.float32),
        mesh=mesh,
        scratch_types=[
            pltpu.VMEM((5, 64), jnp.int32),
            pltpu.VMEM((rows_per_worker, WPOOL), jnp.float32),
            pltpu.VMEM((rpt, WPOOL), jnp.float32),
            pltpu.VMEM_SHARED((ACC2_ROWS, WPOOL), jnp.float32),
        ],
        compiler_params=pltpu.CompilerParams(use_tc_tiling_on_sc=False),
        name="sc_pool_accum",
    )
    def k(d_hbm, pm_hbm, out_hbm, pm_v, dbuf, stage, acc):
        c = lax.axis_index("c")
        s = lax.axis_index("s")
        wid = c * NSUB + s

        zero16 = jnp.zeros((16,), jnp.float32)

        @pl.loop(0, rpt)
        def _zero(i):
            for v in range(nv):
                stage[i, pl.ds(v * 16, 16)] = zero16

        pltpu.sync_copy(stage, acc.at[pl.ds(s * rpt, rpt)])
        pltpu.sync_copy(pm_hbm.at[wid], pm_v)
        pltpu.sync_copy(d_hbm.at[wid], dbuf)
        plsc.subcore_barrier()

        for t in range(5):
            pltpu.sync_copy(dbuf.at[pl.ds(t * 64, 64)], acc.at[pm_v.at[t]], add=True)

        plsc.subcore_barrier()
        rsl = pl.ds(s * rpt, rpt)
        pltpu.sync_copy(acc.at[rsl], stage)
        pltpu.sync_copy(stage, out_hbm.at[c, rsl])

    return k


def _unpool_gather():
    rows_per_worker = NPOOL // NW
    mesh = plsc.VectorSubcoreMesh(core_axis_name="c", subcore_axis_name="s")

    @functools.partial(
        pl.kernel,
        out_type=jax.ShapeDtypeStruct((NPOOL, C0), jnp.float32),
        mesh=mesh,
        scratch_types=[
            pltpu.VMEM((4, 80), jnp.int32),
            pltpu.VMEM((rows_per_worker, C0), jnp.float32),
            pltpu.SemaphoreType.DMA,
        ],
        compiler_params=pltpu.CompilerParams(use_tc_tiling_on_sc=False),
        name="sc_unpool_gather",
    )
    def k(u_hbm, pm_hbm, out_hbm, pm_v, gbuf, sem):
        c = lax.axis_index("c")
        s = lax.axis_index("s")
        wid = c * NSUB + s

        pltpu.sync_copy(pm_hbm.at[wid], pm_v)
        for t in range(4):
            pltpu.async_copy(u_hbm.at[pm_v.at[t]],
                             gbuf.at[pl.ds(t * 80, 80)], sem).wait()
        pltpu.sync_copy(gbuf, out_hbm.at[pl.ds(wid * rows_per_worker,
                                               rows_per_worker)])

    return k


def _lift_bn_relu(x, w, g, b):
    def body(x_ref, w_ref, g_ref, b_ref, h_ref):
        feat = jnp.dot(x_ref[...], w_ref[...], preferred_element_type=jnp.float32)
        mean = jnp.mean(feat, axis=0, keepdims=True)
        var = jnp.mean((feat - mean) ** 2, axis=0, keepdims=True)
        hn = g_ref[...] * (feat - mean) / jnp.sqrt(var + EPS) + b_ref[...]
        h_ref[...] = jnp.maximum(hn, 0.0)

    n = x.shape[0]
    co = w.shape[1]
    return pl.pallas_call(
        body, out_shape=jax.ShapeDtypeStruct((n, co), jnp.float32),
    )(x, w, g.reshape(1, -1), b.reshape(1, -1))


def _build_table(h, wcat, bm):
    n, cin = h.shape
    wn = wcat.shape[1]

    def body(h_ref, w_ref, out_ref):
        out_ref[...] = jnp.dot(h_ref[...], w_ref[...],
                               preferred_element_type=jnp.float32)

    return pl.pallas_call(
        body,
        grid=(n // bm,),
        in_specs=[pl.BlockSpec((bm, cin), lambda i: (i, 0)),
                  pl.BlockSpec((cin, wn), lambda i: (0, 0))],
        out_specs=pl.BlockSpec((bm, wn), lambda i: (i, 0)),
        out_shape=jax.ShapeDtypeStruct((n, wn), jnp.float32),
    )(h, wcat)


def _conv1_norm_down(p0, p1, dT, dw):
    def body(p0_ref, p1_ref, dT_ref, dw_ref, skip_ref, dext_ref):
        acc = p0_ref[...] + p1_ref[...]
        num = acc[:N0]
        dT2 = dT_ref[...]
        deg = dT2[:N0, 0:1] + dT2[:N0, 1:2]
        skip = num / jnp.maximum(deg, 1.0)
        skip_ref[...] = skip
        d = jnp.dot(jnp.maximum(skip, 0.0), dw_ref[...],
                    preferred_element_type=jnp.float32)
        dfull = jnp.concatenate(
            [d, jnp.ones((N0, 1), jnp.float32),
             jnp.zeros((N0, WPOOL - C1 - 1), jnp.float32)], axis=1)
        dext_ref[...] = jnp.concatenate(
            [dfull, jnp.zeros((NPOOL - N0, WPOOL), jnp.float32)], axis=0)

    return pl.pallas_call(
        body,
        out_shape=(jax.ShapeDtypeStruct((N0, C0), jnp.float32),
                   jax.ShapeDtypeStruct((NPOOL, WPOOL), jnp.float32)),
    )(p0, p1, dT, dw)


def _pool_bn_relu(p0, p1, g, b):
    def body(p0_ref, p1_ref, g_ref, b_ref, h_ref):
        acc = p0_ref[...] + p1_ref[...]
        pooled = acc[:, :C1] / jnp.maximum(acc[:, C1:C1 + 1], 1.0)
        mean = jnp.mean(pooled, axis=0, keepdims=True)
        var = jnp.mean((pooled - mean) ** 2, axis=0, keepdims=True)
        hn = g_ref[...] * (pooled - mean) / jnp.sqrt(var + EPS) + b_ref[...]
        h_ref[...] = jnp.maximum(hn, 0.0)

    return pl.pallas_call(
        body, out_shape=jax.ShapeDtypeStruct((N1, C1), jnp.float32),
    )(p0, p1, g.reshape(1, -1), b.reshape(1, -1))


def _conv2_norm_up(p0, p1, dT, uw):
    def body(p0_ref, p1_ref, dT_ref, uw_ref, u_ref):
        acc = p0_ref[...] + p1_ref[...]
        dT2 = dT_ref[...]
        h2 = acc / jnp.maximum(dT2[:, 0:1] + dT2[:, 1:2], 1.0)
        u_ref[...] = jnp.dot(h2, uw_ref[...], preferred_element_type=jnp.float32)

    return pl.pallas_call(
        body, out_shape=jax.ShapeDtypeStruct((N1, C0), jnp.float32),
    )(p0, p1, dT, uw)


def _join_out(skip, up, wa, wb):
    def body(s_ref, u_ref, wa_ref, wb_ref, out_ref):
        out_ref[...] = (
            jnp.dot(jnp.maximum(s_ref[...], 0.0), wa_ref[...],
                    preferred_element_type=jnp.float32)
            + jnp.dot(jnp.maximum(u_ref[...], 0.0), wb_ref[...],
                      preferred_element_type=jnp.float32))

    return pl.pallas_call(
        body, out_shape=jax.ShapeDtypeStruct((N0, C0), jnp.float32),
    )(skip, up, wa, wb)


def _make_wcat(conv_w, width):
    kk, cin, cout = conv_w.shape
    wp = jnp.pad(conv_w, ((0, 0), (0, 0), (0, width - cout)))
    return jnp.transpose(wp, (1, 0, 2)).reshape(cin, kk * width)


@functools.lru_cache(maxsize=None)
def _sc_kernels():
    return (_edge_accum(CH0A, CH0B, W1P, ACC1_ROWS, "sc_edge_accum1"),
            _edge_accum(CH1A, CH1B, W2P, ACC2_ROWS, "sc_edge_accum2"),
            _pool_accum(),
            _unpool_gather())


def _pad_edges(src, kidx, dst, tot_chunks, junk, junk_rows):
    pad = tot_chunks * 128 - src.shape[0]
    jd = junk + (np.arange(pad, dtype=np.int32) % junk_rows)
    srcp = jnp.concatenate([src, jnp.zeros((pad,), jnp.int32)]).reshape(tot_chunks, 128)
    kip = jnp.concatenate([kidx, jnp.zeros((pad,), jnp.int32)]).reshape(tot_chunks, 128)
    dstp = jnp.concatenate([dst, jnp.asarray(jd)]).reshape(tot_chunks, 128)
    return srcp, kip, dstp


def kernel(x, edge_index, kernel_idx, pool_map, edge_index2, kernel_idx2,
           lin0_W, conv1_W, bn1_g, bn1_b, down_W, conv2_W, bn2_g, bn2_b,
           up_W, out_W):
    src0, dst0 = edge_index[0], edge_index[1]
    src1, dst1 = edge_index2[0], edge_index2[1]
    _edge_accum1, _edge_accum2, _pool_accum_k, _unpool_k = _sc_kernels()

    w1cat = _make_wcat(conv1_W, W1P)
    w2cat = _make_wcat(conv2_W, W2P)
    s0p, k0p, d0p = _pad_edges(src0, kernel_idx, dst0, TCH0, ACC1_JUNK, ACC1_ROWS - ACC1_JUNK)
    s1p, k1p, d1p = _pad_edges(src1, kernel_idx2, dst1, TCH1, ACC2_JUNK, ACC2_ROWS - ACC2_JUNK)
    pmp = jnp.concatenate([pool_map, jnp.zeros((NPOOL - N0,), jnp.int32)])
    pm_pool = pmp.reshape(NW, 5, 64)
    pm_up = pmp.reshape(NW, 4, 80)

    h = _lift_bn_relu(x, lin0_W, bn1_g, bn1_b)
    table1 = _build_table(h, w1cat, 1000).reshape(N0 * K, W1P)
    acc1, deg1 = _edge_accum1(table1, s0p, k0p, d0p)
    skip, d_ext = _conv1_norm_down(acc1[0], acc1[1], deg1.T, down_W)

    d_rows = d_ext.reshape(NW, NPOOL // NW, WPOOL)
    paccs = _pool_accum_k(d_rows, pm_pool)
    h2p = _pool_bn_relu(paccs[0, :N1], paccs[1, :N1], bn2_g, bn2_b)

    h2p_pad = jnp.concatenate([h2p, jnp.zeros((2560 - N1, C1), jnp.float32)])
    table2 = _build_table(h2p_pad, w2cat, 512).reshape(2560 * K, W2P)
    acc2, deg2 = _edge_accum2(table2, s1p, k1p, d1p)
    u = _conv2_norm_up(acc2[0, :N1], acc2[1, :N1], deg2.T[:N1], up_W)

    up = _unpool_k(u, pm_up)[:N0]
    out = _join_out(skip, up, out_W[:C0], out_W[C0:])
    return out
